# Initial kernel scaffold; baseline (speedup 1.0000x reference)
import jax
import jax.numpy as jnp
from jax.experimental import pallas as pl


def kernel(x, W_g):
    raise NotImplementedError("write your pallas kernel here")


# speedup vs baseline: 1919.9755x; 1919.9755x over previous
import functools

import jax
import jax.numpy as jnp
from jax import lax
from jax.experimental import pallas as pl
from jax.experimental.pallas import tpu as pltpu
from jax.experimental.pallas import tpu_sc as plsc

DIM = 2048
NEXP = 16
STOK = 4096
CAP = int(1.25 * STOK / NEXP)
TOK_BLK = 512
LANES = 16


def _router_body(x_ref, wt_ref, rnd_ref, e1_ref, e2_ref, g_ref, rc_ref):
    logits = jnp.dot(x_ref[...], wt_ref[...], preferred_element_type=jnp.float32)
    cols = lax.broadcasted_iota(jnp.int32, logits.shape, 1)
    l1 = jnp.max(logits, axis=1, keepdims=True)
    e1 = jnp.min(jnp.where(logits == l1, cols, NEXP), axis=1)
    lm = jnp.where(cols == e1[:, None], -jnp.inf, logits)
    l2 = jnp.max(lm, axis=1, keepdims=True)
    e2 = jnp.min(jnp.where(lm == l2, cols, NEXP), axis=1)
    g = 1.0 / (1.0 + jnp.exp(l2[:, 0] - l1[:, 0]))
    e1_ref[...] = e1
    e2_ref[...] = e2
    g_ref[...] = g
    rc_ref[...] = (rnd_ref[...] < 2.0 * g).astype(jnp.int32)


_router = pl.pallas_call(
    _router_body,
    grid=(STOK // TOK_BLK,),
    in_specs=[
        pl.BlockSpec((TOK_BLK, DIM), lambda i: (i, 0)),
        pl.BlockSpec((DIM, NEXP), lambda i: (0, 0)),
        pl.BlockSpec((TOK_BLK,), lambda i: (i,)),
    ],
    out_specs=[pl.BlockSpec((TOK_BLK,), lambda i: (i,))] * 4,
    out_shape=[
        jax.ShapeDtypeStruct((STOK,), jnp.int32),
        jax.ShapeDtypeStruct((STOK,), jnp.int32),
        jax.ShapeDtypeStruct((STOK,), jnp.float32),
        jax.ShapeDtypeStruct((STOK,), jnp.int32),
    ],
)


@functools.cache
def _build_assign():
    mesh = plsc.VectorSubcoreMesh(core_axis_name="c", subcore_axis_name="s")
    return functools.partial(
        pl.kernel,
        mesh=mesh,
        compiler_params=pltpu.CompilerParams(needs_layout_passes=False),
        out_type=jax.ShapeDtypeStruct((NEXP, STOK), jnp.float32),
        scratch_types=[
            pltpu.VMEM((STOK,), jnp.int32),
            pltpu.VMEM((STOK,), jnp.float32),
            pltpu.VMEM((STOK,), jnp.int32),
            pltpu.VMEM((STOK,), jnp.float32),
        ],
    )(_assign_body)


def _assign_body(e1_hbm, e2_hbm, g_hbm, rc_hbm, out_hbm, idx_v, g_v, rc_v, col_v):
    cid = lax.axis_index("c")
    sid = lax.axis_index("s")
    nchunks = STOK // LANES

    @pl.when(cid == 0)
    def _():
        e = sid
        pltpu.sync_copy(e1_hbm, idx_v)
        pltpu.sync_copy(g_hbm, g_v)
        pltpu.sync_copy(rc_hbm, rc_v)

        def pass1(k, c):
            v = idx_v[pl.ds(k * LANES, LANES)]
            m = v == e
            inc = jnp.where(m, 1, 0)
            pc = jnp.cumsum(inc)
            ok = m & ((pc - inc + c) < CAP)
            gs = g_v[pl.ds(k * LANES, LANES)]
            col_v[pl.ds(k * LANES, LANES)] = jnp.where(ok, gs, 0.0)
            return c + jnp.max(pc)

        n1 = lax.fori_loop(0, nchunks, pass1, 0)
        pltpu.sync_copy(e2_hbm, idx_v)

        def pass2(k, c):
            v = idx_v[pl.ds(k * LANES, LANES)]
            m = (v == e) & (rc_v[pl.ds(k * LANES, LANES)] != 0)
            inc = jnp.where(m, 1, 0)
            pc = jnp.cumsum(inc)
            ok = m & ((pc - inc + c) < CAP)
            gs = g_v[pl.ds(k * LANES, LANES)]
            prev = col_v[pl.ds(k * LANES, LANES)]
            col_v[pl.ds(k * LANES, LANES)] = jnp.where(ok, gs, prev)
            return c + jnp.max(pc)

        lax.fori_loop(0, nchunks, pass2, n1)
        pltpu.sync_copy(col_v, out_hbm.at[e])


def kernel(x, W_g):
    rnd = jax.random.uniform(jax.random.key(42), (x.shape[0],), dtype=jnp.float32)
    e1, e2, g, rc = _router(x, W_g.T, rnd)
    combine_t = _build_assign()(e1, e2, g, rc)
    return combine_t.T


# speedup vs baseline: 2674.4142x; 1.3929x over previous
import functools

import jax
import jax.numpy as jnp
from jax import lax
from jax.experimental import pallas as pl
from jax.experimental.pallas import tpu as pltpu
from jax.experimental.pallas import tpu_sc as plsc

DIM = 2048
NEXP = 16
STOK = 4096
CAP = int(1.25 * STOK / NEXP)
TOK_BLK = 512
LANES = 16


def _router_body(x_ref, wt_ref, mt_ref):
    logits = jnp.dot(x_ref[...], wt_ref[...], preferred_element_type=jnp.float32)
    cols = lax.broadcasted_iota(jnp.int32, logits.shape, 1)
    l1 = jnp.max(logits, axis=1, keepdims=True)
    e1 = jnp.min(jnp.where(logits == l1, cols, NEXP), axis=1, keepdims=True)
    m1 = cols == e1
    lm = jnp.where(m1, -jnp.inf, logits)
    l2 = jnp.max(lm, axis=1, keepdims=True)
    e2 = jnp.min(jnp.where(lm == l2, cols, NEXP), axis=1, keepdims=True)
    m2 = cols == e2
    g = 1.0 / (1.0 + jnp.exp(l2 - l1))
    m = jnp.where(m1, g, jnp.where(m2, -g, 0.0))
    mt_ref[...] = m.T


_router = pl.pallas_call(
    _router_body,
    grid=(STOK // TOK_BLK,),
    in_specs=[
        pl.BlockSpec((TOK_BLK, DIM), lambda i: (i, 0)),
        pl.BlockSpec((DIM, NEXP), lambda i: (0, 0)),
    ],
    out_specs=pl.BlockSpec((NEXP, TOK_BLK), lambda i: (0, i)),
    out_shape=jax.ShapeDtypeStruct((NEXP, STOK), jnp.float32),
)


@functools.cache
def _build_assign():
    mesh = plsc.VectorSubcoreMesh(core_axis_name="c", subcore_axis_name="s")
    return functools.partial(
        pl.kernel,
        mesh=mesh,
        compiler_params=pltpu.CompilerParams(needs_layout_passes=False),
        out_type=jax.ShapeDtypeStruct((NEXP, STOK), jnp.float32),
        scratch_types=[
            pltpu.VMEM((STOK,), jnp.float32),
            pltpu.VMEM((STOK,), jnp.float32),
            pltpu.VMEM((STOK,), jnp.float32),
        ],
    )(_assign_body)


def _assign_body(mt_hbm, rnd_hbm, out_hbm, row_v, rnd_v, col_v):
    cid = lax.axis_index("c")
    sid = lax.axis_index("s")
    nchunks = STOK // LANES

    @pl.when(cid == 0)
    def _():
        e = sid
        pltpu.sync_copy(mt_hbm.at[e], row_v)
        pltpu.sync_copy(rnd_hbm, rnd_v)

        def pass1(k, c):
            cv = row_v[pl.ds(k * LANES, LANES)]
            m = cv > 0.0
            inc = jnp.where(m, 1, 0)
            pc = jnp.cumsum(inc)
            ok = m & ((pc - inc + c) < CAP)
            col_v[pl.ds(k * LANES, LANES)] = jnp.where(ok, cv, 0.0)
            return c + plsc.all_reduce_population_count(m)

        n1 = lax.fori_loop(0, nchunks, pass1, jnp.zeros((LANES,), jnp.int32))

        def pass2(k, c):
            cv = row_v[pl.ds(k * LANES, LANES)]
            m = (cv < 0.0) & (rnd_v[pl.ds(k * LANES, LANES)] < -2.0 * cv)
            inc = jnp.where(m, 1, 0)
            pc = jnp.cumsum(inc)
            ok = m & ((pc - inc + c) < CAP)
            prev = col_v[pl.ds(k * LANES, LANES)]
            col_v[pl.ds(k * LANES, LANES)] = jnp.where(ok, -cv, prev)
            return c + plsc.all_reduce_population_count(m)

        lax.fori_loop(0, nchunks, pass2, n1)
        pltpu.sync_copy(col_v, out_hbm.at[e])


def kernel(x, W_g):
    with jax.ensure_compile_time_eval():
        rnd = jax.random.uniform(jax.random.key(42), (x.shape[0],), dtype=jnp.float32)
    mt = _router(x, W_g.T)
    combine_t = _build_assign()(mt, rnd)
    return combine_t.T


# speedup vs baseline: 2678.8143x; 1.0016x over previous
import functools

import jax
import jax.numpy as jnp
from jax import lax
from jax.experimental import pallas as pl
from jax.experimental.pallas import tpu as pltpu
from jax.experimental.pallas import tpu_sc as plsc

DIM = 2048
NEXP = 16
STOK = 4096
CAP = int(1.25 * STOK / NEXP)
TOK_BLK = 512
LANES = 16


def _router_body(x_ref, wt_ref, mt_ref):
    logits = jnp.dot(x_ref[...], wt_ref[...], preferred_element_type=jnp.float32)
    cols = lax.broadcasted_iota(jnp.int32, logits.shape, 1)
    l1 = jnp.max(logits, axis=1, keepdims=True)
    e1 = jnp.min(jnp.where(logits == l1, cols, NEXP), axis=1, keepdims=True)
    m1 = cols == e1
    lm = jnp.where(m1, -jnp.inf, logits)
    l2 = jnp.max(lm, axis=1, keepdims=True)
    e2 = jnp.min(jnp.where(lm == l2, cols, NEXP), axis=1, keepdims=True)
    m2 = cols == e2
    g = 1.0 / (1.0 + jnp.exp(l2 - l1))
    m = jnp.where(m1, g, jnp.where(m2, -g, 0.0))
    mt_ref[...] = m.T


_router = pl.pallas_call(
    _router_body,
    grid=(STOK // TOK_BLK,),
    in_specs=[
        pl.BlockSpec((TOK_BLK, DIM), lambda i: (i, 0)),
        pl.BlockSpec((DIM, NEXP), lambda i: (0, 0)),
    ],
    out_specs=pl.BlockSpec((NEXP, TOK_BLK), lambda i: (0, i)),
    out_shape=jax.ShapeDtypeStruct((NEXP, STOK), jnp.float32),
)


@functools.cache
def _build_assign():
    mesh = plsc.VectorSubcoreMesh(core_axis_name="c", subcore_axis_name="s")
    return functools.partial(
        pl.kernel,
        mesh=mesh,
        compiler_params=pltpu.CompilerParams(needs_layout_passes=False),
        out_type=jax.ShapeDtypeStruct((NEXP, STOK), jnp.float32),
        scratch_types=[
            pltpu.VMEM((STOK,), jnp.float32),
            pltpu.VMEM((STOK,), jnp.float32),
            pltpu.VMEM((STOK,), jnp.float32),
        ],
    )(_assign_body)


def _assign_body(mt_hbm, rnd_hbm, out_hbm, row_v, rnd_v, col_v):
    cid = lax.axis_index("c")
    sid = lax.axis_index("s")
    nchunks = STOK // LANES

    @pl.when(cid == 0)
    def _():
        e = sid
        pltpu.sync_copy(mt_hbm.at[e], row_v)
        pltpu.sync_copy(rnd_hbm, rnd_v)

        def pass1(k, c):
            cv = row_v[pl.ds(k * LANES, LANES)]
            m = cv > 0.0
            inc = jnp.where(m, 1, 0)
            pc = jnp.cumsum(inc)
            ok = m & ((pc - inc + c) < CAP)
            col_v[pl.ds(k * LANES, LANES)] = jnp.where(ok, cv, 0.0)
            return c + plsc.all_reduce_population_count(m)

        n1 = lax.fori_loop(0, nchunks, pass1, jnp.zeros((LANES,), jnp.int32))

        def pass2(k, c):
            cv = row_v[pl.ds(k * LANES, LANES)]
            m = (cv < 0.0) & (rnd_v[pl.ds(k * LANES, LANES)] < -2.0 * cv)
            inc = jnp.where(m, 1, 0)
            pc = jnp.cumsum(inc)
            ok = m & ((pc - inc + c) < CAP)
            prev = col_v[pl.ds(k * LANES, LANES)]
            col_v[pl.ds(k * LANES, LANES)] = jnp.where(ok, -cv, prev)
            return c + plsc.all_reduce_population_count(m)

        lax.fori_loop(0, nchunks, pass2, n1)
        pltpu.sync_copy(col_v, out_hbm.at[e])


def kernel(x, W_g):
    try:
        with jax.ensure_compile_time_eval():
            rnd = jax.random.uniform(jax.random.key(42), (x.shape[0],), dtype=jnp.float32)
    except Exception:
        rnd = jax.random.uniform(jax.random.key(42), (x.shape[0],), dtype=jnp.float32)
    mt = _router(x, W_g.T)
    combine_t = _build_assign()(mt, rnd)
    return combine_t.T
